# baseline (speedup 1.0000x reference)
import functools

import jax
import jax.numpy as jnp
from jax import lax
from jax.experimental import pallas as pl
from jax.experimental.pallas import tpu as pltpu
from jax.experimental.pallas import tpu_sc as plsc

LANES = 16


@functools.lru_cache(maxsize=None)
def _build_emb(B: int, D: int, V: int):
    info = plsc.get_sparse_core_info()
    NC, NS = info.num_cores, info.num_subcores
    NW = NC * NS
    assert B % (NW * LANES) == 0
    bpw = B // NW
    ngrp = bpw // LANES
    mesh = plsc.VectorSubcoreMesh(core_axis_name="c", subcore_axis_name="s")

    @functools.partial(
        pl.kernel,
        mesh=mesh,
        out_type=jax.ShapeDtypeStruct((B, D), jnp.float32),
        scratch_types=[
            pltpu.VMEM((bpw,), jnp.int32),
            pltpu.VMEM((V, D), jnp.float32),
            pltpu.SemaphoreType.DMA,
            pltpu.SemaphoreType.DMA,
        ],
    )
    def emb(idx_hbm, table_hbm, out_hbm, idx_v, tab_v, sem, stage_sem):
        wid = lax.axis_index("s") * NC + lax.axis_index("c")
        base = wid * bpw
        tab_cp = pltpu.async_copy(table_hbm, tab_v, stage_sem)
        idx_cp = pltpu.async_copy(idx_hbm.at[pl.ds(base, bpw)], idx_v, stage_sem)
        tab_cp.wait()
        idx_cp.wait()

        def grp_body(g, carry):
            v16 = idx_v[pl.ds(g * LANES, LANES)]
            for k in range(LANES):
                row = v16[k]
                pltpu.async_copy(
                    tab_v.at[row], out_hbm.at[base + g * LANES + k], sem
                )
            return carry

        lax.fori_loop(0, ngrp, grp_body, 0)

        def drain(q, carry):
            pltpu.make_async_copy(
                tab_v.at[pl.ds(0, LANES)],
                out_hbm.at[pl.ds(base, LANES)],
                sem,
            ).wait()
            return carry

        lax.fori_loop(0, ngrp, drain, 0)

    return emb


def kernel(seqs, W_embed):
    batch, seq = seqs.shape
    V, D = W_embed.shape
    idx = seqs.reshape(-1).astype(jnp.int32)
    emb = _build_emb(batch * seq, D, V)
    out = emb(idx, W_embed)
    return out.reshape(batch, seq, D)


# speedup vs baseline: 1.0060x; 1.0060x over previous
import functools

import jax
import jax.numpy as jnp
from jax import lax
from jax.experimental import pallas as pl
from jax.experimental.pallas import tpu as pltpu
from jax.experimental.pallas import tpu_sc as plsc

LANES = 16


@functools.lru_cache(maxsize=None)
def _build_emb(B: int, D: int, V: int):
    info = plsc.get_sparse_core_info()
    NC, NS = info.num_cores, info.num_subcores
    NW = NC * NS
    assert B % (NW * LANES) == 0
    bpw = B // NW
    ngrp = bpw // LANES
    mesh = plsc.VectorSubcoreMesh(core_axis_name="c", subcore_axis_name="s")

    @functools.partial(
        pl.kernel,
        mesh=mesh,
        out_type=jax.ShapeDtypeStruct((B, D), jnp.float32),
        scratch_types=[
            pltpu.VMEM((bpw,), jnp.int32),
            pltpu.VMEM((V, D), jnp.float32),
            pltpu.SemaphoreType.DMA,
        ],
    )
    def emb(idx_hbm, table_hbm, out_hbm, idx_v, tab_v, sem):
        wid = lax.axis_index("s") * NC + lax.axis_index("c")
        base = wid * bpw
        pltpu.sync_copy(table_hbm, tab_v)
        pltpu.sync_copy(idx_hbm.at[pl.ds(base, bpw)], idx_v)

        def grp_body(g, carry):
            v16 = idx_v[pl.ds(g * LANES, LANES)]
            for k in range(LANES):
                row = v16[k]
                pltpu.async_copy(
                    tab_v.at[row], out_hbm.at[base + g * LANES + k], sem
                )
            return carry

        lax.fori_loop(0, ngrp, grp_body, 0)

        def drain(q, carry):
            pltpu.make_async_copy(
                tab_v.at[pl.ds(0, LANES)],
                out_hbm.at[pl.ds(base, LANES)],
                sem,
            ).wait()
            return carry

        lax.fori_loop(0, ngrp, drain, 0)

    return emb


def kernel(seqs, W_embed):
    batch, seq = seqs.shape
    V, D = W_embed.shape
    idx = seqs.reshape(-1).astype(jnp.int32)
    emb = _build_emb(batch * seq, D, V)
    out = emb(idx, W_embed)
    return out.reshape(batch, seq, D)


# speedup vs baseline: 1.0380x; 1.0319x over previous
import functools

import jax
import jax.numpy as jnp
from jax import lax
from jax.experimental import pallas as pl
from jax.experimental.pallas import tpu as pltpu
from jax.experimental.pallas import tpu_sc as plsc

LANES = 16


@functools.lru_cache(maxsize=None)
def _build_emb(B: int, D: int, V: int):
    info = plsc.get_sparse_core_info()
    NC, NS = info.num_cores, info.num_subcores
    NW = NC * NS
    assert B % (NW * LANES) == 0
    bpw = B // NW
    ngrp = bpw // LANES
    mesh = plsc.VectorSubcoreMesh(core_axis_name="c", subcore_axis_name="s")

    @functools.partial(
        pl.kernel,
        mesh=mesh,
        out_type=jax.ShapeDtypeStruct((B, D), jnp.float32),
        scratch_types=[
            pltpu.VMEM((bpw,), jnp.int32),
            pltpu.VMEM((V, D), jnp.float32),
            pltpu.VMEM_SHARED((V, D), jnp.float32),
            pltpu.SemaphoreType.DMA,
        ],
    )
    def emb(idx_hbm, table_hbm, out_hbm, idx_v, tab_v, tab_sh, sem):
        wid = lax.axis_index("s") * NC + lax.axis_index("c")
        base = wid * bpw

        @pl.when(lax.axis_index("s") == 0)
        def _():
            pltpu.sync_copy(table_hbm, tab_sh)

        pltpu.sync_copy(idx_hbm.at[pl.ds(base, bpw)], idx_v)
        plsc.subcore_barrier()
        pltpu.sync_copy(tab_sh, tab_v)

        def grp_body(g, carry):
            v16 = idx_v[pl.ds(g * LANES, LANES)]
            for k in range(LANES):
                row = v16[k]
                pltpu.async_copy(
                    tab_v.at[row], out_hbm.at[base + g * LANES + k], sem
                )
            return carry

        lax.fori_loop(0, ngrp, grp_body, 0)

        def drain(q, carry):
            pltpu.make_async_copy(
                tab_v.at[pl.ds(0, LANES)],
                out_hbm.at[pl.ds(base, LANES)],
                sem,
            ).wait()
            return carry

        lax.fori_loop(0, ngrp, drain, 0)

    return emb


def kernel(seqs, W_embed):
    batch, seq = seqs.shape
    V, D = W_embed.shape
    idx = seqs.reshape(-1).astype(jnp.int32)
    emb = _build_emb(batch * seq, D, V)
    out = emb(idx, W_embed)
    return out.reshape(batch, seq, D)
